# baseline (speedup 1.0000x reference)
import jax
import jax.numpy as jnp
from jax import lax
from jax.experimental import pallas as pl
from jax.experimental.pallas import tpu as pltpu
from jax.experimental.pallas import tpu_sc as plsc

B = 4096
G = 100
OD = 16
NW = 32
BT = B // NW
NCHUNK = BT // 16
ROW = G * OD


def _splat(x, l):
    return lax.gather(
        x,
        jnp.zeros((16, 1), jnp.int32) + l,
        lax.GatherDimensionNumbers(
            offset_dims=(), collapsed_slice_dims=(0,), start_index_map=(0,)),
        (1,),
        mode=lax.GatherScatterMode.PROMISE_IN_BOUNDS,
    )


def _body(xt_hbm, tab_hbm, out_hbm, a_buf, tab_buf, out_buf, sem0, sem1):
    wid = lax.axis_index("s") * 2 + lax.axis_index("c")
    pltpu.sync_copy(xt_hbm.at[:, pl.ds(wid * BT, BT)], a_buf)
    pltpu.sync_copy(tab_hbm, tab_buf)

    def do_g(g, par16, sem):
        @plsc.parallel_loop(0, NCHUNK, unroll=2)
        def c_body(c):
            col = c * 16
            va0 = a_buf[3 * g, pl.ds(col, 16)]
            va1 = a_buf[3 * g + 1, pl.ds(col, 16)]
            va2 = a_buf[3 * g + 2, pl.ds(col, 16)]
            vmin = jnp.minimum(jnp.minimum(va0, va1), va2)
            vmax = jnp.maximum(jnp.maximum(va0, va1), va2)
            vmid = jnp.maximum(jnp.minimum(va0, va1),
                               jnp.minimum(jnp.maximum(va0, va1), va2))
            c0 = vmin
            c1 = vmid - vmin
            c2 = vmax - vmid
            pmin = jnp.where(va0 == vmin, jnp.int32(1),
                             jnp.where(va1 == vmin, jnp.int32(2),
                                       jnp.int32(4)))
            pmax = jnp.where(va2 == vmax, jnp.int32(4),
                             jnp.where(va1 == vmax, jnp.int32(2),
                                       jnp.int32(1)))
            gbase = g * 136
            idx_mid = gbase + 119 - pmin * 17
            idx_max = gbase + pmax * 17
            row7 = tab_buf[pl.ds(gbase + 119, 16)]
            for l in range(16):
                s7 = jnp.broadcast_to(row7[l], (16,))
                smid = plsc.load_gather(tab_buf, [idx_mid + l])
                smax = plsc.load_gather(tab_buf, [idx_max + l])
                o = c0 * s7 + c1 * smid + c2 * smax
                out_buf[par16 + l, pl.ds(col, 16)] = o

        pltpu.async_copy(
            out_buf.at[pl.ds(par16, 16)],
            out_hbm.at[pl.ds(g * 16, 16), pl.ds(wid * BT, BT)],
            sem,
        )

    def wait_g(g, par16, sem):
        pltpu.make_async_copy(
            out_buf.at[pl.ds(par16, 16)],
            out_hbm.at[pl.ds(g * 16, 16), pl.ds(wid * BT, BT)],
            sem,
        ).wait()

    def pair_body(gg, carry):
        g_even = gg * 2
        g_odd = gg * 2 + 1

        @pl.when(gg >= 1)
        def _w0():
            wait_g(g_even - 2, 0, sem0)

        do_g(g_even, 0, sem0)

        @pl.when(gg >= 1)
        def _w1():
            wait_g(g_odd - 2, 16, sem1)

        do_g(g_odd, 16, sem1)
        return carry

    lax.fori_loop(0, G // 2, pair_body, 0)
    wait_g(G - 2, 0, sem0)
    wait_g(G - 1, 16, sem1)


@jax.jit
def kernel(X, params):
    xt = X.T
    tab = jnp.pad(params.reshape(G * 8, OD), ((0, 0), (0, 1))).reshape(G * 8 * 17)
    run = pl.kernel(
        _body,
        out_type=jax.ShapeDtypeStruct((ROW, B), jnp.float32),
        mesh=plsc.VectorSubcoreMesh(core_axis_name="c", subcore_axis_name="s"),
        compiler_params=pltpu.CompilerParams(
            needs_layout_passes=False, use_tc_tiling_on_sc=True),
        scratch_types=[
            pltpu.VMEM((3 * G, BT), jnp.float32),
            pltpu.VMEM((G * 8 * 17,), jnp.float32),
            pltpu.VMEM((32, BT), jnp.float32),
            pltpu.SemaphoreType.DMA,
            pltpu.SemaphoreType.DMA,
        ],
    )
    out = run(xt, tab)
    return out.T


# speedup vs baseline: 1.0885x; 1.0885x over previous
import jax
import jax.numpy as jnp
from jax import lax
from jax.experimental import pallas as pl
from jax.experimental.pallas import tpu as pltpu
from jax.experimental.pallas import tpu_sc as plsc

B = 4096
G = 100
OD = 16
NW = 32
BT = B // NW
NCHUNK = BT // 16
ROW = G * OD


def _splat(x, l):
    return lax.gather(
        x,
        jnp.zeros((16, 1), jnp.int32) + l,
        lax.GatherDimensionNumbers(
            offset_dims=(), collapsed_slice_dims=(0,), start_index_map=(0,)),
        (1,),
        mode=lax.GatherScatterMode.PROMISE_IN_BOUNDS,
    )


def _body(xt_hbm, tab_hbm, out_hbm, a_buf, tab_buf, out_buf, sem0, sem1):
    wid = lax.axis_index("s") * 2 + lax.axis_index("c")
    pltpu.sync_copy(xt_hbm.at[:, pl.ds(wid * BT, BT)], a_buf)
    pltpu.sync_copy(tab_hbm, tab_buf)

    def do_g(g, par16, sem):
        def c_body(c, carry):
            col = c * 16
            va0 = a_buf[3 * g, pl.ds(col, 16)]
            va1 = a_buf[3 * g + 1, pl.ds(col, 16)]
            va2 = a_buf[3 * g + 2, pl.ds(col, 16)]
            vmin = jnp.minimum(jnp.minimum(va0, va1), va2)
            vmax = jnp.maximum(jnp.maximum(va0, va1), va2)
            vmid = jnp.maximum(jnp.minimum(va0, va1),
                               jnp.minimum(jnp.maximum(va0, va1), va2))
            c0 = vmin
            c1 = vmid - vmin
            c2 = vmax - vmid
            pmin = jnp.where((va0 <= va1) & (va0 <= va2), jnp.int32(1),
                             jnp.where(va1 <= va2, jnp.int32(2),
                                       jnp.int32(4)))
            pmax = jnp.where((va2 >= va0) & (va2 >= va1), jnp.int32(4),
                             jnp.where(va1 >= va0, jnp.int32(2),
                                       jnp.int32(1)))
            gbase = g * 136
            idx_mid = gbase + 119 - pmin * 17
            idx_max = gbase + pmax * 17
            row7 = tab_buf[pl.ds(gbase + 119, 16)]

            @plsc.parallel_loop(0, 16, unroll=4)
            def l_body(l):
                s7 = _splat(row7, l)
                smid = plsc.load_gather(tab_buf, [idx_mid + l])
                smax = plsc.load_gather(tab_buf, [idx_max + l])
                o = c0 * s7 + c1 * smid + c2 * smax
                out_buf[par16 + l, pl.ds(col, 16)] = o

            return carry

        lax.fori_loop(0, NCHUNK, c_body, 0)
        pltpu.async_copy(
            out_buf.at[pl.ds(par16, 16)],
            out_hbm.at[pl.ds(g * 16, 16), pl.ds(wid * BT, BT)],
            sem,
        )

    def wait_g(g, par16, sem):
        pltpu.make_async_copy(
            out_buf.at[pl.ds(par16, 16)],
            out_hbm.at[pl.ds(g * 16, 16), pl.ds(wid * BT, BT)],
            sem,
        ).wait()

    def pair_body(gg, carry):
        g_even = gg * 2
        g_odd = gg * 2 + 1

        @pl.when(gg >= 1)
        def _w0():
            wait_g(g_even - 2, 0, sem0)

        do_g(g_even, 0, sem0)

        @pl.when(gg >= 1)
        def _w1():
            wait_g(g_odd - 2, 16, sem1)

        do_g(g_odd, 16, sem1)
        return carry

    lax.fori_loop(0, G // 2, pair_body, 0)
    wait_g(G - 2, 0, sem0)
    wait_g(G - 1, 16, sem1)


@jax.jit
def kernel(X, params):
    xt = X.T
    tab = jnp.pad(params.reshape(G * 8, OD), ((0, 0), (0, 1))).reshape(G * 8 * 17)
    run = pl.kernel(
        _body,
        out_type=jax.ShapeDtypeStruct((ROW, B), jnp.float32),
        mesh=plsc.VectorSubcoreMesh(core_axis_name="c", subcore_axis_name="s"),
        compiler_params=pltpu.CompilerParams(
            needs_layout_passes=False, use_tc_tiling_on_sc=True),
        scratch_types=[
            pltpu.VMEM((3 * G, BT), jnp.float32),
            pltpu.VMEM((G * 8 * 17,), jnp.float32),
            pltpu.VMEM((32, BT), jnp.float32),
            pltpu.SemaphoreType.DMA,
            pltpu.SemaphoreType.DMA,
        ],
    )
    out = run(xt, tab)
    return out.T


# speedup vs baseline: 1.2669x; 1.1639x over previous
import jax
import jax.numpy as jnp
from jax import lax
from jax.experimental import pallas as pl
from jax.experimental.pallas import tpu as pltpu
from jax.experimental.pallas import tpu_sc as plsc

B = 4096
G = 100
OD = 16
NW = 32
BT = B // NW
NCHUNK = BT // 16
ROW = G * OD


def _splat(x, l):
    return lax.gather(
        x,
        jnp.zeros((16, 1), jnp.int32) + l,
        lax.GatherDimensionNumbers(
            offset_dims=(), collapsed_slice_dims=(0,), start_index_map=(0,)),
        (1,),
        mode=lax.GatherScatterMode.PROMISE_IN_BOUNDS,
    )


def _body(xt_hbm, tab_hbm, out_hbm, a_buf, tab_buf, out_buf, sem0, sem1):
    wid = lax.axis_index("s") * 2 + lax.axis_index("c")
    pltpu.sync_copy(xt_hbm.at[:, pl.ds(wid * BT, BT)], a_buf)
    pltpu.sync_copy(tab_hbm, tab_buf)

    def do_g(g, par16, sem):
        gbase = g * 136
        row7 = tab_buf[pl.ds(gbase + 119, 16)]

        def coefs(col):
            va0 = a_buf[3 * g, pl.ds(col, 16)]
            va1 = a_buf[3 * g + 1, pl.ds(col, 16)]
            va2 = a_buf[3 * g + 2, pl.ds(col, 16)]
            vmin = jnp.minimum(jnp.minimum(va0, va1), va2)
            vmax = jnp.maximum(jnp.maximum(va0, va1), va2)
            vmid = jnp.maximum(jnp.minimum(va0, va1),
                               jnp.minimum(jnp.maximum(va0, va1), va2))
            pmin = jnp.where((va0 <= va1) & (va0 <= va2), jnp.int32(1),
                             jnp.where(va1 <= va2, jnp.int32(2),
                                       jnp.int32(4)))
            pmax = jnp.where((va2 >= va0) & (va2 >= va1), jnp.int32(4),
                             jnp.where(va1 >= va0, jnp.int32(2),
                                       jnp.int32(1)))
            return (vmin, vmid - vmin, vmax - vmid,
                    gbase + 119 - pmin * 17, gbase + pmax * 17)

        def c_body(ci, carry):
            cola = ci * 32
            colb = cola + 16
            c0a, c1a, c2a, idx_mid_a, idx_max_a = coefs(cola)
            c0b, c1b, c2b, idx_mid_b, idx_max_b = coefs(colb)

            @plsc.parallel_loop(0, 16, unroll=4)
            def l_body(l):
                s7 = _splat(row7, l)
                smid_a = plsc.load_gather(tab_buf, [idx_mid_a + l])
                smax_a = plsc.load_gather(tab_buf, [idx_max_a + l])
                oa = c0a * s7 + c1a * smid_a + c2a * smax_a
                out_buf[par16 + l, pl.ds(cola, 16)] = oa
                smid_b = plsc.load_gather(tab_buf, [idx_mid_b + l])
                smax_b = plsc.load_gather(tab_buf, [idx_max_b + l])
                ob = c0b * s7 + c1b * smid_b + c2b * smax_b
                out_buf[par16 + l, pl.ds(colb, 16)] = ob

            return carry

        lax.fori_loop(0, NCHUNK // 2, c_body, 0)
        pltpu.async_copy(
            out_buf.at[pl.ds(par16, 16)],
            out_hbm.at[pl.ds(g * 16, 16), pl.ds(wid * BT, BT)],
            sem,
        )

    def wait_g(g, par16, sem):
        pltpu.make_async_copy(
            out_buf.at[pl.ds(par16, 16)],
            out_hbm.at[pl.ds(g * 16, 16), pl.ds(wid * BT, BT)],
            sem,
        ).wait()

    def pair_body(gg, carry):
        g_even = gg * 2
        g_odd = gg * 2 + 1

        @pl.when(gg >= 1)
        def _w0():
            wait_g(g_even - 2, 0, sem0)

        do_g(g_even, 0, sem0)

        @pl.when(gg >= 1)
        def _w1():
            wait_g(g_odd - 2, 16, sem1)

        do_g(g_odd, 16, sem1)
        return carry

    lax.fori_loop(0, G // 2, pair_body, 0)
    wait_g(G - 2, 0, sem0)
    wait_g(G - 1, 16, sem1)


@jax.jit
def kernel(X, params):
    xt = X.T
    tab = jnp.pad(params.reshape(G * 8, OD), ((0, 0), (0, 1))).reshape(G * 8 * 17)
    run = pl.kernel(
        _body,
        out_type=jax.ShapeDtypeStruct((ROW, B), jnp.float32),
        mesh=plsc.VectorSubcoreMesh(core_axis_name="c", subcore_axis_name="s"),
        compiler_params=pltpu.CompilerParams(
            needs_layout_passes=False, use_tc_tiling_on_sc=True),
        scratch_types=[
            pltpu.VMEM((3 * G, BT), jnp.float32),
            pltpu.VMEM((G * 8 * 17,), jnp.float32),
            pltpu.VMEM((32, BT), jnp.float32),
            pltpu.SemaphoreType.DMA,
            pltpu.SemaphoreType.DMA,
        ],
    )
    out = run(xt, tab)
    return out.T


# speedup vs baseline: 1.3345x; 1.0533x over previous
import jax
import jax.numpy as jnp
from jax import lax
from jax.experimental import pallas as pl
from jax.experimental.pallas import tpu as pltpu
from jax.experimental.pallas import tpu_sc as plsc

B = 4096
G = 100
OD = 16
NW = 32
BT = B // NW
NCHUNK = BT // 16
ROW = G * OD


def _splat(x, l):
    return lax.gather(
        x,
        jnp.zeros((16, 1), jnp.int32) + l,
        lax.GatherDimensionNumbers(
            offset_dims=(), collapsed_slice_dims=(0,), start_index_map=(0,)),
        (1,),
        mode=lax.GatherScatterMode.PROMISE_IN_BOUNDS,
    )


def _body(xt_hbm, tab_hbm, out_hbm, a_buf, tab_buf, out_buf, sem0, sem1):
    wid = lax.axis_index("s") * 2 + lax.axis_index("c")
    pltpu.sync_copy(xt_hbm.at[:, pl.ds(wid * BT, BT)], a_buf)
    pltpu.sync_copy(tab_hbm, tab_buf)

    def do_g(g, par16, sem):
        gbase = g * 136
        row7 = tab_buf[pl.ds(gbase + 119, 16)]

        def coefs(col):
            va0 = a_buf[3 * g, pl.ds(col, 16)]
            va1 = a_buf[3 * g + 1, pl.ds(col, 16)]
            va2 = a_buf[3 * g + 2, pl.ds(col, 16)]
            vmin = jnp.minimum(jnp.minimum(va0, va1), va2)
            vmax = jnp.maximum(jnp.maximum(va0, va1), va2)
            vmid = jnp.maximum(jnp.minimum(va0, va1),
                               jnp.minimum(jnp.maximum(va0, va1), va2))
            pmin = jnp.where((va0 <= va1) & (va0 <= va2), jnp.int32(1),
                             jnp.where(va1 <= va2, jnp.int32(2),
                                       jnp.int32(4)))
            pmax = jnp.where((va2 >= va0) & (va2 >= va1), jnp.int32(4),
                             jnp.where(va1 >= va0, jnp.int32(2),
                                       jnp.int32(1)))
            return (vmin, vmid - vmin, vmax - vmid,
                    gbase + 119 - pmin * 17, gbase + pmax * 17)

        def c_body(ci, carry):
            cols = [ci * 64 + k * 16 for k in range(4)]
            cc = [coefs(col) for col in cols]

            @plsc.parallel_loop(0, 16, unroll=2)
            def l_body(l):
                s7 = _splat(row7, l)
                for (c0, c1, c2, idx_mid, idx_max), col in zip(cc, cols):
                    smid = plsc.load_gather(tab_buf, [idx_mid + l])
                    smax = plsc.load_gather(tab_buf, [idx_max + l])
                    o = c0 * s7 + c1 * smid + c2 * smax
                    out_buf[par16 + l, pl.ds(col, 16)] = o

            return carry

        lax.fori_loop(0, NCHUNK // 4, c_body, 0)
        pltpu.async_copy(
            out_buf.at[pl.ds(par16, 16)],
            out_hbm.at[pl.ds(g * 16, 16), pl.ds(wid * BT, BT)],
            sem,
        )

    def wait_g(g, par16, sem):
        pltpu.make_async_copy(
            out_buf.at[pl.ds(par16, 16)],
            out_hbm.at[pl.ds(g * 16, 16), pl.ds(wid * BT, BT)],
            sem,
        ).wait()

    def pair_body(gg, carry):
        g_even = gg * 2
        g_odd = gg * 2 + 1

        @pl.when(gg >= 1)
        def _w0():
            wait_g(g_even - 2, 0, sem0)

        do_g(g_even, 0, sem0)

        @pl.when(gg >= 1)
        def _w1():
            wait_g(g_odd - 2, 16, sem1)

        do_g(g_odd, 16, sem1)
        return carry

    lax.fori_loop(0, G // 2, pair_body, 0)
    wait_g(G - 2, 0, sem0)
    wait_g(G - 1, 16, sem1)


@jax.jit
def kernel(X, params):
    xt = X.T
    tab = jnp.pad(params.reshape(G * 8, OD), ((0, 0), (0, 1))).reshape(G * 8 * 17)
    run = pl.kernel(
        _body,
        out_type=jax.ShapeDtypeStruct((ROW, B), jnp.float32),
        mesh=plsc.VectorSubcoreMesh(core_axis_name="c", subcore_axis_name="s"),
        compiler_params=pltpu.CompilerParams(
            needs_layout_passes=False, use_tc_tiling_on_sc=True),
        scratch_types=[
            pltpu.VMEM((3 * G, BT), jnp.float32),
            pltpu.VMEM((G * 8 * 17,), jnp.float32),
            pltpu.VMEM((32, BT), jnp.float32),
            pltpu.SemaphoreType.DMA,
            pltpu.SemaphoreType.DMA,
        ],
    )
    out = run(xt, tab)
    return out.T
